# baseline (speedup 1.0000x reference)
import jax
import jax.numpy as jnp
from jax.experimental import pallas as pl
from jax.experimental.pallas import tpu as pltpu

_BM = 80
_NBUF = 8


def _rgcn_kernel(a_hbm, xf_ref, w_ref, o_ref, bufs, b_ref, sems):
    n = a_hbm.shape[0]
    nchunk = n // _BM

    def mk(j):
        return pltpu.make_async_copy(
            a_hbm.at[pl.ds(j * _BM, _BM), :],
            bufs.at[j % _NBUF],
            sems.at[j % _NBUF],
        )

    for j in range(_NBUF - 1):
        mk(j).start()

    wp = jnp.transpose(w_ref[...], (1, 0, 2)).reshape(
        w_ref.shape[0] * w_ref.shape[1], w_ref.shape[2]
    )
    b_ref[...] = jnp.dot(xf_ref[...], wp, preferred_element_type=jnp.float32)
    b = b_ref[...]

    for j in range(nchunk):
        if j + _NBUF - 1 < nchunk:
            mk(j + _NBUF - 1).start()
        mk(j).wait()
        o_ref[pl.ds(j * _BM, _BM), :] = jnp.dot(
            bufs.at[j % _NBUF][...], b, preferred_element_type=jnp.float32
        )


def kernel(a, x, w):
    n = a.shape[0]
    i_sz, r_sz = x.shape[1], x.shape[2]
    o_sz = w.shape[2]
    xflat = x.reshape(n, i_sz * r_sz)

    return pl.pallas_call(
        _rgcn_kernel,
        in_specs=[
            pl.BlockSpec(memory_space=pltpu.MemorySpace.HBM),
            pl.BlockSpec(memory_space=pltpu.MemorySpace.VMEM),
            pl.BlockSpec(memory_space=pltpu.MemorySpace.VMEM),
        ],
        out_specs=pl.BlockSpec(memory_space=pltpu.MemorySpace.VMEM),
        out_shape=jax.ShapeDtypeStruct((n, o_sz), jnp.float32),
        scratch_shapes=[
            pltpu.VMEM((_NBUF, _BM, n), jnp.float32),
            pltpu.VMEM((n, o_sz), jnp.float32),
            pltpu.SemaphoreType.DMA((_NBUF,)),
        ],
    )(a, xflat, w)


# speedup vs baseline: 1.0066x; 1.0066x over previous
import jax
import jax.numpy as jnp
from jax.experimental import pallas as pl
from jax.experimental.pallas import tpu as pltpu

_BM = 400


def _rgcn_kernel(a_ref, xf_ref, w_ref, o_ref, b_ref):
    @pl.when(pl.program_id(0) == 0)
    def _():
        wp = jnp.transpose(w_ref[...], (1, 0, 2)).reshape(
            w_ref.shape[0] * w_ref.shape[1], w_ref.shape[2]
        )
        b_ref[...] = jnp.dot(xf_ref[...], wp, preferred_element_type=jnp.float32)

    o_ref[...] = jnp.dot(a_ref[...], b_ref[...], preferred_element_type=jnp.float32)


def kernel(a, x, w):
    n = a.shape[0]
    i_sz, r_sz = x.shape[1], x.shape[2]
    o_sz = w.shape[2]
    xflat = x.reshape(n, i_sz * r_sz)

    grid = (n // _BM,)
    return pl.pallas_call(
        _rgcn_kernel,
        grid=grid,
        in_specs=[
            pl.BlockSpec((_BM, n), lambda i: (i, 0)),
            pl.BlockSpec((n, i_sz * r_sz), lambda i: (0, 0)),
            pl.BlockSpec((r_sz, i_sz, o_sz), lambda i: (0, 0, 0)),
        ],
        out_specs=pl.BlockSpec((_BM, o_sz), lambda i: (i, 0)),
        out_shape=jax.ShapeDtypeStruct((n, o_sz), jnp.float32),
        scratch_shapes=[pltpu.VMEM((n, o_sz), jnp.float32)],
    )(a, xflat, w)
